# baseline (speedup 1.0000x reference)
import functools

import jax
import jax.numpy as jnp
from jax import lax
from jax.experimental import pallas as pl
from jax.experimental.pallas import tpu as pltpu
from jax.experimental.pallas import tpu_sc as plsc

T = 512
H = 1024
I = 512
E = 16
A = 2 * T
B = 128
NB = 23
NBP = NB * B


def _router_body(x_ref, gw_ref, gb_ref, pw_ref, meta_ref):
    x = x_ref[...]
    logits = lax.dot_general(x, gw_ref[...], (((1,), (0,)), ((), ())),
                             preferred_element_type=jnp.float32) + gb_ref[...]
    z = logits - jnp.max(logits, axis=1, keepdims=True)
    ez = jnp.exp(z)
    rw = ez / jnp.sum(ez, axis=1, keepdims=True)
    lane = lax.broadcasted_iota(jnp.int32, (T, E), 1)
    m1 = jnp.max(rw, axis=1, keepdims=True)
    e0 = jnp.min(jnp.where(rw == m1, lane, E), axis=1, keepdims=True)
    oh0 = (lane == e0)
    rwx = jnp.where(oh0, -jnp.inf, rw)
    m2 = jnp.max(rwx, axis=1, keepdims=True)
    e1 = jnp.min(jnp.where(rwx == m2, lane, E), axis=1, keepdims=True)
    oh1 = (lane == e1)
    s12 = m1 + m2
    w0 = m1 / s12
    w1 = m2 / s12
    oh0f = oh0.astype(jnp.float32)
    oh1f = oh1.astype(jnp.float32)
    oh = oh0f + oh1f
    ri = lax.broadcasted_iota(jnp.int32, (T, T), 0)
    ci = lax.broadcasted_iota(jnp.int32, (T, T), 1)
    ls = (ri > ci).astype(jnp.float32)
    cnt = lax.dot_general(ls, oh, (((1,), (0,)), ((), ())),
                          preferred_element_type=jnp.float32)
    tot_r = jnp.sum(oh, axis=0, keepdims=True)
    nbe_r = jnp.floor((tot_r + (B - 1.0)) / B)
    uei = lax.broadcasted_iota(jnp.int32, (E, E), 0)
    uec = lax.broadcasted_iota(jnp.int32, (E, E), 1)
    us = (uei < uec).astype(jnp.float32)
    excl_r = lax.dot_general(nbe_r, us, (((1,), (0,)), ((), ())),
                             preferred_element_type=jnp.float32)
    base = excl_r * float(B) + cnt
    p0 = jnp.sum(oh0f * base, axis=1, keepdims=True)
    p1 = jnp.sum(oh1f * base, axis=1, keepdims=True)
    pw_ref[...] = jnp.concatenate([p0, p1, w0, w1], axis=1)
    tot_c = lax.dot_general(oh, jnp.ones((T, 1), jnp.float32),
                            (((0,), (0,)), ((), ())),
                            preferred_element_type=jnp.float32)
    nbe_c = jnp.floor((tot_c + (B - 1.0)) / B)
    lci = lax.broadcasted_iota(jnp.int32, (E, E), 0)
    lcc = lax.broadcasted_iota(jnp.int32, (E, E), 1)
    lsE = (lci > lcc).astype(jnp.float32)
    excl_c = lax.dot_general(lsE, nbe_c, (((1,), (0,)), ((), ())),
                             preferred_element_type=jnp.float32)
    cum_incl = excl_c + nbe_c
    nb_total = jnp.sum(nbe_r)
    biota = lax.broadcasted_iota(jnp.int32, (E, 32), 1).astype(jnp.float32)
    bc = jnp.minimum(biota, nb_total - 1.0)
    blk = jnp.sum((cum_incl <= bc).astype(jnp.int32), axis=0, keepdims=True)
    lane32 = lax.broadcasted_iota(jnp.int32, (1, 32), 1)
    meta_ref[...] = jnp.where(lane32 == 31, nb_total.astype(jnp.int32), blk)


def _scatter_body(pos_hbm, wgt_hbm, tv_hbm, zi_hbm, zf_hbm, tok_out, wgt_out,
                  pos_v, win_v, tv_v, tok_v, wp_v):
    c = lax.axis_index("c")
    s = lax.axis_index("s")

    @pl.when((c == 0) & (s == 0))
    def _work():
        pltpu.sync_copy(pos_hbm, pos_v)
        pltpu.sync_copy(wgt_hbm, win_v)
        pltpu.sync_copy(tv_hbm, tv_v)
        pltpu.sync_copy(zi_hbm, tok_v)
        pltpu.sync_copy(zf_hbm, wp_v)

        def chunk(ci, carry):
            idx = pos_v[pl.ds(ci * 16, 16)]
            wv = win_v[pl.ds(ci * 16, 16)]
            tv = tv_v[pl.ds(ci * 16, 16)]
            plsc.store_scatter(tok_v, [idx], tv)
            plsc.store_scatter(wp_v, [idx], wv)
            return carry

        lax.fori_loop(0, A // 16, chunk, 0)
        pltpu.sync_copy(tok_v, tok_out)
        pltpu.sync_copy(wp_v, wgt_out)


def _ffn_body(meta_ref, tok_ref, wgt_ref, x_ref, w1_ref, w2_ref, w3_ref,
              out_ref):
    b = pl.program_id(0)
    nb = meta_ref[31]

    @pl.when(b == 0)
    def _init():
        out_ref[...] = jnp.zeros_like(out_ref)

    @pl.when(b < nb)
    def _block():
        tok = tok_ref[0]
        ti = lax.broadcasted_iota(jnp.int32, (T, B), 0)
        gt = (tok == ti).astype(jnp.float32)
        xb = lax.dot_general(gt, x_ref[...], (((0,), (0,)), ((), ())),
                             preferred_element_type=jnp.float32)
        h1 = lax.dot_general(xb, w1_ref[0], (((1,), (1,)), ((), ())),
                             preferred_element_type=jnp.float32)
        h3 = lax.dot_general(xb, w3_ref[0], (((1,), (1,)), ((), ())),
                             preferred_element_type=jnp.float32)
        hh = h1 * jax.nn.sigmoid(h1) * h3
        y = lax.dot_general(hh, w2_ref[0], (((1,), (1,)), ((), ())),
                            preferred_element_type=jnp.float32)
        gw = gt * wgt_ref[0]
        out_ref[...] += lax.dot_general(gw, y, (((1,), (0,)), ((), ())),
                                        preferred_element_type=jnp.float32)


def _router_call(x, gate_w, gb2):
    return pl.pallas_call(
        _router_body,
        in_specs=[
            pl.BlockSpec((T, H), lambda: (0, 0)),
            pl.BlockSpec((H, E), lambda: (0, 0)),
            pl.BlockSpec((1, E), lambda: (0, 0)),
        ],
        out_specs=[
            pl.BlockSpec((T, 4), lambda: (0, 0)),
            pl.BlockSpec((1, 32), lambda: (0, 0)),
        ],
        out_shape=[
            jax.ShapeDtypeStruct((T, 4), jnp.float32),
            jax.ShapeDtypeStruct((1, 32), jnp.int32),
        ],
    )(x, gate_w, gb2)


def _scatter_call(pos_i, wgt_f, tvals, zi, zf):
    mesh = plsc.VectorSubcoreMesh(core_axis_name="c", subcore_axis_name="s")
    f = pl.kernel(
        _scatter_body,
        mesh=mesh,
        out_type=[
            jax.ShapeDtypeStruct((NBP,), jnp.int32),
            jax.ShapeDtypeStruct((NBP,), jnp.float32),
        ],
        scratch_types=[
            pltpu.VMEM((A,), jnp.int32),
            pltpu.VMEM((A,), jnp.float32),
            pltpu.VMEM((A,), jnp.int32),
            pltpu.VMEM((NBP,), jnp.int32),
            pltpu.VMEM((NBP,), jnp.float32),
        ],
        compiler_params=pltpu.CompilerParams(needs_layout_passes=False),
    )
    return f(pos_i, wgt_f, tvals, zi, zf)


def _ffn_call(meta, tok3, wgt3, x, w1, w2, w3):
    grid_spec = pltpu.PrefetchScalarGridSpec(
        num_scalar_prefetch=1,
        grid=(NB,),
        in_specs=[
            pl.BlockSpec((1, 1, B), lambda b, m: (b, 0, 0)),
            pl.BlockSpec((1, 1, B), lambda b, m: (b, 0, 0)),
            pl.BlockSpec((T, H), lambda b, m: (0, 0)),
            pl.BlockSpec((1, I, H), lambda b, m: (m[b], 0, 0)),
            pl.BlockSpec((1, H, I), lambda b, m: (m[b], 0, 0)),
            pl.BlockSpec((1, I, H), lambda b, m: (m[b], 0, 0)),
        ],
        out_specs=pl.BlockSpec((T, H), lambda b, m: (0, 0)),
    )
    return pl.pallas_call(
        _ffn_body,
        grid_spec=grid_spec,
        out_shape=jax.ShapeDtypeStruct((T, H), jnp.float32),
        compiler_params=pltpu.CompilerParams(
            dimension_semantics=("arbitrary",)),
    )(meta, tok3, wgt3, x, w1, w2, w3)


@jax.jit
def kernel(x, gate_w, gate_b, w1, w2, w3):
    gb2 = gate_b.reshape(1, E)
    pw, meta = _router_call(x, gate_w, gb2)
    pos_i = pw[:, :2].reshape(A).astype(jnp.int32)
    wgt_f = pw[:, 2:4].reshape(A)
    tvals = jnp.arange(A, dtype=jnp.int32) // 2
    zi = jnp.zeros((NBP,), jnp.int32)
    zf = jnp.zeros((NBP,), jnp.float32)
    tok_pad, wgt_pad = _scatter_call(pos_i, wgt_f, tvals, zi, zf)
    tok3 = tok_pad.reshape(NB, 1, B)
    wgt3 = wgt_pad.reshape(NB, 1, B)
    return _ffn_call(meta.reshape(32), tok3, wgt3, x, w1, w2, w3)


# speedup vs baseline: 1.5301x; 1.5301x over previous
import jax
import jax.numpy as jnp
from jax import lax
from jax.experimental import pallas as pl
from jax.experimental.pallas import tpu as pltpu

T = 512
H = 1024
I = 512
E = 16
A = 2 * T
B = 128
NB = 23
NBP = NB * B


def _router_body(x_ref, gw_ref, gb_ref, pw_ref, meta_ref):
    x = x_ref[...]
    logits = lax.dot_general(x, gw_ref[...], (((1,), (0,)), ((), ())),
                             preferred_element_type=jnp.float32) + gb_ref[...]
    z = logits - jnp.max(logits, axis=1, keepdims=True)
    ez = jnp.exp(z)
    rw = ez / jnp.sum(ez, axis=1, keepdims=True)
    lane = lax.broadcasted_iota(jnp.int32, (T, E), 1)
    m1 = jnp.max(rw, axis=1, keepdims=True)
    e0 = jnp.min(jnp.where(rw == m1, lane, E), axis=1, keepdims=True)
    oh0 = (lane == e0)
    rwx = jnp.where(oh0, -jnp.inf, rw)
    m2 = jnp.max(rwx, axis=1, keepdims=True)
    e1 = jnp.min(jnp.where(rwx == m2, lane, E), axis=1, keepdims=True)
    oh1 = (lane == e1)
    s12 = m1 + m2
    w0 = m1 / s12
    w1 = m2 / s12
    oh0f = oh0.astype(jnp.float32)
    oh1f = oh1.astype(jnp.float32)
    oh = oh0f + oh1f
    ri = lax.broadcasted_iota(jnp.int32, (T, T), 0)
    ci = lax.broadcasted_iota(jnp.int32, (T, T), 1)
    ls = (ri > ci).astype(jnp.float32)
    cnt = lax.dot_general(ls, oh, (((1,), (0,)), ((), ())),
                          preferred_element_type=jnp.float32)
    tot_r = jnp.sum(oh, axis=0, keepdims=True)
    nbe_r = jnp.floor((tot_r + (B - 1.0)) / B)
    uei = lax.broadcasted_iota(jnp.int32, (E, E), 0)
    uec = lax.broadcasted_iota(jnp.int32, (E, E), 1)
    us = (uei < uec).astype(jnp.float32)
    excl_r = lax.dot_general(nbe_r, us, (((1,), (0,)), ((), ())),
                             preferred_element_type=jnp.float32)
    base = excl_r * float(B) + cnt
    p0 = jnp.sum(oh0f * base, axis=1, keepdims=True)
    p1 = jnp.sum(oh1f * base, axis=1, keepdims=True)
    pw_ref[...] = jnp.concatenate([p0, p1, w0, w1], axis=1)
    tot_c = lax.dot_general(oh, jnp.ones((T, 1), jnp.float32),
                            (((0,), (0,)), ((), ())),
                            preferred_element_type=jnp.float32)
    nbe_c = jnp.floor((tot_c + (B - 1.0)) / B)
    lci = lax.broadcasted_iota(jnp.int32, (E, E), 0)
    lcc = lax.broadcasted_iota(jnp.int32, (E, E), 1)
    lsE = (lci > lcc).astype(jnp.float32)
    excl_c = lax.dot_general(lsE, nbe_c, (((1,), (0,)), ((), ())),
                             preferred_element_type=jnp.float32)
    cum_incl = excl_c + nbe_c
    nb_total = jnp.sum(nbe_r)
    biota = lax.broadcasted_iota(jnp.int32, (E, 32), 1).astype(jnp.float32)
    bc = jnp.minimum(biota, nb_total - 1.0)
    blk = jnp.sum((cum_incl <= bc).astype(jnp.int32), axis=0, keepdims=True)
    lane32 = lax.broadcasted_iota(jnp.int32, (1, 32), 1)
    meta_ref[...] = jnp.where(lane32 == 31, nb_total.astype(jnp.int32), blk)


def _ffn_body(meta_ref, pw_ref, x_ref, w1_ref, w2_ref, w3_ref, out_ref):
    b = pl.program_id(0)
    nb = meta_ref[31]

    @pl.when(b == 0)
    def _init():
        out_ref[...] = jnp.zeros_like(out_ref)

    @pl.when(b < nb)
    def _block():
        pw = pw_ref[...]
        pos0 = pw[:, 0:1]
        pos1 = pw[:, 1:2]
        bi = (b * B + lax.broadcasted_iota(jnp.int32, (T, B), 1)
              ).astype(jnp.float32)
        g0 = (pos0 == bi).astype(jnp.float32)
        g1 = (pos1 == bi).astype(jnp.float32)
        gt = g0 + g1
        gw = pw[:, 2:3] * g0 + pw[:, 3:4] * g1
        xb = lax.dot_general(gt, x_ref[...], (((0,), (0,)), ((), ())),
                             preferred_element_type=jnp.float32)
        h1 = lax.dot_general(xb, w1_ref[0], (((1,), (1,)), ((), ())),
                             preferred_element_type=jnp.float32)
        h3 = lax.dot_general(xb, w3_ref[0], (((1,), (1,)), ((), ())),
                             preferred_element_type=jnp.float32)
        hh = h1 * jax.nn.sigmoid(h1) * h3
        y = lax.dot_general(hh, w2_ref[0], (((1,), (1,)), ((), ())),
                            preferred_element_type=jnp.float32)
        out_ref[...] += lax.dot_general(gw, y, (((1,), (0,)), ((), ())),
                                        preferred_element_type=jnp.float32)


def _router_call(x, gate_w, gb2):
    return pl.pallas_call(
        _router_body,
        in_specs=[
            pl.BlockSpec((T, H), lambda: (0, 0)),
            pl.BlockSpec((H, E), lambda: (0, 0)),
            pl.BlockSpec((1, E), lambda: (0, 0)),
        ],
        out_specs=[
            pl.BlockSpec((T, 4), lambda: (0, 0)),
            pl.BlockSpec((1, 32), lambda: (0, 0)),
        ],
        out_shape=[
            jax.ShapeDtypeStruct((T, 4), jnp.float32),
            jax.ShapeDtypeStruct((1, 32), jnp.int32),
        ],
    )(x, gate_w, gb2)


def _ffn_call(meta, pw, x, w1, w2, w3):
    grid_spec = pltpu.PrefetchScalarGridSpec(
        num_scalar_prefetch=1,
        grid=(NB,),
        in_specs=[
            pl.BlockSpec((T, 4), lambda b, m: (0, 0)),
            pl.BlockSpec((T, H), lambda b, m: (0, 0)),
            pl.BlockSpec((1, I, H), lambda b, m: (m[b], 0, 0)),
            pl.BlockSpec((1, H, I), lambda b, m: (m[b], 0, 0)),
            pl.BlockSpec((1, I, H), lambda b, m: (m[b], 0, 0)),
        ],
        out_specs=pl.BlockSpec((T, H), lambda b, m: (0, 0)),
    )
    return pl.pallas_call(
        _ffn_body,
        grid_spec=grid_spec,
        out_shape=jax.ShapeDtypeStruct((T, H), jnp.float32),
        compiler_params=pltpu.CompilerParams(
            dimension_semantics=("arbitrary",)),
    )(meta, pw, x, w1, w2, w3)


@jax.jit
def kernel(x, gate_w, gate_b, w1, w2, w3):
    gb2 = gate_b.reshape(1, E)
    pw, meta = _router_call(x, gate_w, gb2)
    return _ffn_call(meta.reshape(32), pw, x, w1, w2, w3)


# speedup vs baseline: 1.5304x; 1.0002x over previous
import jax
import jax.numpy as jnp
from jax import lax
from jax.experimental import pallas as pl
from jax.experimental.pallas import tpu as pltpu

T = 512
H = 1024
I = 512
E = 16
A = 2 * T
B = 128
NB = 23
NBP = NB * B


def _router_body(x_ref, gw_ref, gb_ref, pw_ref, meta_ref):
    x = x_ref[...]
    logits = lax.dot_general(x, gw_ref[...], (((1,), (0,)), ((), ())),
                             preferred_element_type=jnp.float32) + gb_ref[...]
    z = logits - jnp.max(logits, axis=1, keepdims=True)
    ez = jnp.exp(z)
    rw = ez / jnp.sum(ez, axis=1, keepdims=True)
    lane = lax.broadcasted_iota(jnp.int32, (T, E), 1)
    m1 = jnp.max(rw, axis=1, keepdims=True)
    e0 = jnp.min(jnp.where(rw == m1, lane, E), axis=1, keepdims=True)
    oh0 = (lane == e0)
    rwx = jnp.where(oh0, -jnp.inf, rw)
    m2 = jnp.max(rwx, axis=1, keepdims=True)
    e1 = jnp.min(jnp.where(rwx == m2, lane, E), axis=1, keepdims=True)
    oh1 = (lane == e1)
    s12 = m1 + m2
    w0 = m1 / s12
    w1 = m2 / s12
    oh0f = oh0.astype(jnp.float32)
    oh1f = oh1.astype(jnp.float32)
    oh = oh0f + oh1f
    ri = lax.broadcasted_iota(jnp.int32, (T, T), 0)
    ci = lax.broadcasted_iota(jnp.int32, (T, T), 1)
    ls = (ri > ci).astype(jnp.float32)
    cnt = lax.dot_general(ls, oh, (((1,), (0,)), ((), ())),
                          preferred_element_type=jnp.float32)
    tot_r = jnp.sum(oh, axis=0, keepdims=True)
    nbe_r = jnp.floor((tot_r + (B - 1.0)) / B)
    uei = lax.broadcasted_iota(jnp.int32, (E, E), 0)
    uec = lax.broadcasted_iota(jnp.int32, (E, E), 1)
    us = (uei < uec).astype(jnp.float32)
    excl_r = lax.dot_general(nbe_r, us, (((1,), (0,)), ((), ())),
                             preferred_element_type=jnp.float32)
    base = excl_r * float(B) + cnt
    p0 = jnp.sum(oh0f * base, axis=1, keepdims=True)
    p1 = jnp.sum(oh1f * base, axis=1, keepdims=True)
    pw_ref[...] = jnp.concatenate([p0, p1, w0, w1], axis=1)
    tot_c = lax.dot_general(oh, jnp.ones((T, 1), jnp.float32),
                            (((0,), (0,)), ((), ())),
                            preferred_element_type=jnp.float32)
    nbe_c = jnp.floor((tot_c + (B - 1.0)) / B)
    lci = lax.broadcasted_iota(jnp.int32, (E, E), 0)
    lcc = lax.broadcasted_iota(jnp.int32, (E, E), 1)
    lsE = (lci > lcc).astype(jnp.float32)
    excl_c = lax.dot_general(lsE, nbe_c, (((1,), (0,)), ((), ())),
                             preferred_element_type=jnp.float32)
    cum_incl = excl_c + nbe_c
    nb_total = jnp.sum(nbe_r)
    biota = lax.broadcasted_iota(jnp.int32, (E, 32), 1).astype(jnp.float32)
    bc = jnp.minimum(biota, nb_total - 1.0)
    blk = jnp.sum((cum_incl <= bc).astype(jnp.int32), axis=0, keepdims=True)
    lane32 = lax.broadcasted_iota(jnp.int32, (1, 32), 1)
    meta_ref[...] = jnp.where(lane32 == 31, nb_total.astype(jnp.int32), blk)


def _ffn_body(meta_ref, pw_ref, x_ref, w1_ref, w2_ref, w3_ref, out_ref):
    b = pl.program_id(0)
    nb = meta_ref[31]

    @pl.when(b == 0)
    def _init():
        out_ref[...] = jnp.zeros_like(out_ref)

    @pl.when(b < nb)
    def _block():
        pw = pw_ref[...]
        pos0 = pw[:, 0:1]
        pos1 = pw[:, 1:2]
        bi = (b * B + lax.broadcasted_iota(jnp.int32, (T, B), 1)
              ).astype(jnp.float32)
        bf = jnp.bfloat16
        g0 = (pos0 == bi).astype(bf)
        g1 = (pos1 == bi).astype(bf)
        gt = g0 + g1
        gw = pw[:, 2:3].astype(bf) * g0 + pw[:, 3:4].astype(bf) * g1
        xb = lax.dot_general(gt, x_ref[...].astype(bf), (((0,), (0,)), ((), ())),
                             preferred_element_type=jnp.float32
                             ).astype(bf)
        h1 = lax.dot_general(xb, w1_ref[0].astype(bf), (((1,), (1,)), ((), ())),
                             preferred_element_type=jnp.float32)
        h3 = lax.dot_general(xb, w3_ref[0].astype(bf), (((1,), (1,)), ((), ())),
                             preferred_element_type=jnp.float32)
        hh = (h1 * jax.nn.sigmoid(h1) * h3).astype(bf)
        y = lax.dot_general(hh, w2_ref[0].astype(bf), (((1,), (1,)), ((), ())),
                            preferred_element_type=jnp.float32
                            ).astype(bf)
        out_ref[...] += lax.dot_general(gw, y, (((1,), (0,)), ((), ())),
                                        preferred_element_type=jnp.float32)


def _router_call(x, gate_w, gb2):
    return pl.pallas_call(
        _router_body,
        in_specs=[
            pl.BlockSpec((T, H), lambda: (0, 0)),
            pl.BlockSpec((H, E), lambda: (0, 0)),
            pl.BlockSpec((1, E), lambda: (0, 0)),
        ],
        out_specs=[
            pl.BlockSpec((T, 4), lambda: (0, 0)),
            pl.BlockSpec((1, 32), lambda: (0, 0)),
        ],
        out_shape=[
            jax.ShapeDtypeStruct((T, 4), jnp.float32),
            jax.ShapeDtypeStruct((1, 32), jnp.int32),
        ],
    )(x, gate_w, gb2)


def _ffn_call(meta, pw, x, w1, w2, w3):
    grid_spec = pltpu.PrefetchScalarGridSpec(
        num_scalar_prefetch=1,
        grid=(NB,),
        in_specs=[
            pl.BlockSpec((T, 4), lambda b, m: (0, 0)),
            pl.BlockSpec((T, H), lambda b, m: (0, 0)),
            pl.BlockSpec((1, I, H), lambda b, m: (m[b], 0, 0)),
            pl.BlockSpec((1, H, I), lambda b, m: (m[b], 0, 0)),
            pl.BlockSpec((1, I, H), lambda b, m: (m[b], 0, 0)),
        ],
        out_specs=pl.BlockSpec((T, H), lambda b, m: (0, 0)),
    )
    return pl.pallas_call(
        _ffn_body,
        grid_spec=grid_spec,
        out_shape=jax.ShapeDtypeStruct((T, H), jnp.float32),
        compiler_params=pltpu.CompilerParams(
            dimension_semantics=("arbitrary",)),
    )(meta, pw, x, w1, w2, w3)


@jax.jit
def kernel(x, gate_w, gate_b, w1, w2, w3):
    gb2 = gate_b.reshape(1, E)
    pw, meta = _router_call(x, gate_w, gb2)
    return _ffn_call(meta.reshape(32), pw, x, w1, w2, w3)


# speedup vs baseline: 1.6290x; 1.0644x over previous
import jax
import jax.numpy as jnp
from jax import lax
from jax.experimental import pallas as pl
from jax.experimental.pallas import tpu as pltpu

T = 512
H = 1024
I = 512
E = 16


def _moe_body(x_ref, gw_ref, gb_ref, w1_ref, w2_ref, w3_ref, out_ref,
              comb_ref, xb_ref):
    e = pl.program_id(0)
    bf = jnp.bfloat16

    @pl.when(e == 0)
    def _router():
        x = x_ref[...]
        logits = lax.dot_general(
            x, gw_ref[...], (((1,), (0,)), ((), ())),
            preferred_element_type=jnp.float32) + gb_ref[...]
        z = logits - jnp.max(logits, axis=1, keepdims=True)
        ez = jnp.exp(z)
        rw = ez / jnp.sum(ez, axis=1, keepdims=True)
        lane = lax.broadcasted_iota(jnp.int32, (T, E), 1)
        m1 = jnp.max(rw, axis=1, keepdims=True)
        e0 = jnp.min(jnp.where(rw == m1, lane, E), axis=1, keepdims=True)
        oh0 = (lane == e0)
        rwx = jnp.where(oh0, -jnp.inf, rw)
        m2 = jnp.max(rwx, axis=1, keepdims=True)
        e1 = jnp.min(jnp.where(rwx == m2, lane, E), axis=1, keepdims=True)
        oh1 = (lane == e1)
        s12 = m1 + m2
        comb_ref[...] = (jnp.where(oh0, m1 / s12, 0.0)
                         + jnp.where(oh1, m2 / s12, 0.0))
        xb_ref[...] = x.astype(bf)
        out_ref[...] = jnp.zeros_like(out_ref)

    xb = xb_ref[...]
    h1 = lax.dot_general(xb, w1_ref[0].astype(bf), (((1,), (1,)), ((), ())),
                         preferred_element_type=jnp.float32)
    h3 = lax.dot_general(xb, w3_ref[0].astype(bf), (((1,), (1,)), ((), ())),
                         preferred_element_type=jnp.float32)
    hh = (h1 * jax.nn.sigmoid(h1) * h3).astype(bf)
    y = lax.dot_general(hh, w2_ref[0].astype(bf), (((1,), (1,)), ((), ())),
                        preferred_element_type=jnp.float32)
    lane = lax.broadcasted_iota(jnp.int32, (T, E), 1)
    ce = jnp.sum(jnp.where(lane == e, comb_ref[...], 0.0), axis=1,
                 keepdims=True)
    out_ref[...] += ce * y


@jax.jit
def kernel(x, gate_w, gate_b, w1, w2, w3):
    gb2 = gate_b.reshape(1, E)
    return pl.pallas_call(
        _moe_body,
        grid=(E,),
        in_specs=[
            pl.BlockSpec((T, H), lambda e: (0, 0)),
            pl.BlockSpec((H, E), lambda e: (0, 0)),
            pl.BlockSpec((1, E), lambda e: (0, 0)),
            pl.BlockSpec((1, I, H), lambda e: (e, 0, 0)),
            pl.BlockSpec((1, H, I), lambda e: (e, 0, 0)),
            pl.BlockSpec((1, I, H), lambda e: (e, 0, 0)),
        ],
        out_specs=pl.BlockSpec((T, H), lambda e: (0, 0)),
        out_shape=jax.ShapeDtypeStruct((T, H), jnp.float32),
        scratch_shapes=[pltpu.VMEM((T, E), jnp.float32),
                        pltpu.VMEM((T, H), jnp.bfloat16)],
        compiler_params=pltpu.CompilerParams(
            dimension_semantics=("arbitrary",)),
    )(x, gate_w, gb2, w1, w2, w3)


# speedup vs baseline: 1.7439x; 1.0705x over previous
import jax
import jax.numpy as jnp
from jax import lax
from jax.experimental import pallas as pl
from jax.experimental.pallas import tpu as pltpu

T = 512
H = 1024
I = 512
E = 16
EPG = 2
G = E // EPG


def _moe_body(x_ref, gw_ref, gb_ref, w1_ref, w2_ref, w3_ref, out_ref,
              comb_ref):
    g = pl.program_id(0)

    @pl.when(g == 0)
    def _router():
        x = x_ref[...]
        logits = lax.dot_general(
            x, gw_ref[...], (((1,), (0,)), ((), ())),
            preferred_element_type=jnp.float32) + gb_ref[...]
        z = logits - jnp.max(logits, axis=1, keepdims=True)
        ez = jnp.exp(z)
        rw = ez / jnp.sum(ez, axis=1, keepdims=True)
        lane = lax.broadcasted_iota(jnp.int32, (T, E), 1)
        m1 = jnp.max(rw, axis=1, keepdims=True)
        e0 = jnp.min(jnp.where(rw == m1, lane, E), axis=1, keepdims=True)
        oh0 = (lane == e0)
        rwx = jnp.where(oh0, -jnp.inf, rw)
        m2 = jnp.max(rwx, axis=1, keepdims=True)
        e1 = jnp.min(jnp.where(rwx == m2, lane, E), axis=1, keepdims=True)
        oh1 = (lane == e1)
        s12 = m1 + m2
        comb_ref[...] = (jnp.where(oh0, m1 / s12, 0.0)
                         + jnp.where(oh1, m2 / s12, 0.0))
        out_ref[...] = jnp.zeros_like(out_ref)

    x = x_ref[...]
    lane = lax.broadcasted_iota(jnp.int32, (T, E), 1)
    comb = comb_ref[...]
    ys = []
    for sub in range(EPG):
        h1 = lax.dot_general(x, w1_ref[sub], (((1,), (1,)), ((), ())),
                             preferred_element_type=jnp.float32)
        h3 = lax.dot_general(x, w3_ref[sub], (((1,), (1,)), ((), ())),
                             preferred_element_type=jnp.float32)
        hh = h1 * jax.nn.sigmoid(h1) * h3
        ce = jnp.sum(jnp.where(lane == g * EPG + sub, comb, 0.0), axis=1,
                     keepdims=True)
        hc = ce * hh
        ys.append(lax.dot_general(hc, w2_ref[sub], (((1,), (1,)), ((), ())),
                                  preferred_element_type=jnp.float32))
    out_ref[...] += ys[0] + ys[1]


@jax.jit
def kernel(x, gate_w, gate_b, w1, w2, w3):
    gb2 = gate_b.reshape(1, E)
    return pl.pallas_call(
        _moe_body,
        grid=(G,),
        in_specs=[
            pl.BlockSpec((T, H), lambda g: (0, 0)),
            pl.BlockSpec((H, E), lambda g: (0, 0)),
            pl.BlockSpec((1, E), lambda g: (0, 0)),
            pl.BlockSpec((EPG, I, H), lambda g: (g, 0, 0)),
            pl.BlockSpec((EPG, H, I), lambda g: (g, 0, 0)),
            pl.BlockSpec((EPG, I, H), lambda g: (g, 0, 0)),
        ],
        out_specs=pl.BlockSpec((T, H), lambda g: (0, 0)),
        out_shape=jax.ShapeDtypeStruct((T, H), jnp.float32),
        scratch_shapes=[pltpu.VMEM((T, E), jnp.float32)],
        compiler_params=pltpu.CompilerParams(
            dimension_semantics=("arbitrary",)),
    )(x, gate_w, gb2, w1, w2, w3)
